# baseline (speedup 1.0000x reference)
import jax
import jax.numpy as jnp
from jax.experimental import pallas as pl
from jax.experimental.pallas import tpu as pltpu

_N = 10000
_F = 128
_R = 400
_NB = _N // _R


def _gcn2_body(s1_ref, s2_ref, wt_ref, b_ref, pw_ref, adj_ref,
               h1_ref, h2_ref, ps_ref, f_scr):
    i = pl.program_id(0)

    @pl.when(i == 0)
    def _():
        wt = wt_ref[...]
        f_scr[:, :_F] = jnp.dot(s1_ref[...], wt,
                                preferred_element_type=jnp.float32)
        f_scr[:, _F:] = jnp.dot(s2_ref[...], wt,
                                preferred_element_type=jnp.float32)

    acc = jnp.dot(adj_ref[...], f_scr[...],
                  precision=jax.lax.Precision.DEFAULT,
                  preferred_element_type=jnp.float32)
    acc = acc + b_ref[...]
    h = jnp.where(acc > 0, acc, acc * pw_ref[...])
    h1 = h[:, :_F]
    h1_ref[0] = h1
    h2_ref[0] = h[:, _F:]
    ps_ref[0] = jnp.sum(h1, axis=0, keepdims=True)


def _disc_body(ps_ref, wb_ref, bb_ref, h1_ref, h2_ref, sc1_ref, sc2_ref):
    tot = jnp.sum(ps_ref[...], axis=0, keepdims=True)
    c = jax.nn.sigmoid(tot * (1.0 / _N))
    v = jax.lax.dot_general(c, wb_ref[...], (((1,), (1,)), ((), ())),
                            preferred_element_type=jnp.float32)
    sc1_ref[...] = jnp.sum(h1_ref[0] * v, axis=1, keepdims=True) + bb_ref[...]
    sc2_ref[...] = jnp.sum(h2_ref[0] * v, axis=1, keepdims=True) + bb_ref[...]


def kernel(cc_label, seq1, seq2, adj, sparse, msk, samp_bias1, samp_bias2,
           W_fc, b_gcn, prelu_w, W_bil, b_bil):
    s1 = seq1[0]
    s2 = seq2[0]
    A = adj[0]
    wt = W_fc.T
    b2 = jnp.concatenate([b_gcn, b_gcn])[None, :]
    pw = prelu_w.reshape(1, 1)
    bb = b_bil.reshape(1, 1)

    h1, h2, psums = pl.pallas_call(
        _gcn2_body,
        grid=(_NB,),
        in_specs=[
            pl.BlockSpec((_N, _F), lambda i: (0, 0)),
            pl.BlockSpec((_N, _F), lambda i: (0, 0)),
            pl.BlockSpec((_F, _F), lambda i: (0, 0)),
            pl.BlockSpec((1, 2 * _F), lambda i: (0, 0)),
            pl.BlockSpec((1, 1), lambda i: (0, 0)),
            pl.BlockSpec((_R, _N), lambda i: (i, 0)),
        ],
        out_specs=[
            pl.BlockSpec((1, _R, _F), lambda i: (0, i, 0)),
            pl.BlockSpec((1, _R, _F), lambda i: (0, i, 0)),
            pl.BlockSpec((1, 1, _F), lambda i: (i, 0, 0)),
        ],
        out_shape=[
            jax.ShapeDtypeStruct((1, _N, _F), jnp.float32),
            jax.ShapeDtypeStruct((1, _N, _F), jnp.float32),
            jax.ShapeDtypeStruct((_NB, 1, _F), jnp.float32),
        ],
        scratch_shapes=[pltpu.VMEM((_N, 2 * _F), jnp.float32)],
    )(s1, s2, wt, b2, pw, A)

    sc1, sc2 = pl.pallas_call(
        _disc_body,
        out_shape=[
            jax.ShapeDtypeStruct((_N, 1), jnp.float32),
            jax.ShapeDtypeStruct((_N, 1), jnp.float32),
        ],
    )(psums.reshape(_NB, _F), W_bil[0], bb, h1, h2)

    ret = jnp.concatenate([sc1[:, 0][None, :] + samp_bias1,
                           sc2[:, 0][None, :] + samp_bias2], axis=1)
    return (ret, h1, h2)


# speedup vs baseline: 1.0007x; 1.0007x over previous
import jax
import jax.numpy as jnp
from jax.experimental import pallas as pl
from jax.experimental.pallas import tpu as pltpu

_N = 10000
_F = 128
_R = 400
_NB = _N // _R


def _dgi_body(s1_ref, s2_ref, wt_ref, b_ref, pw_ref, wb_ref, bb_ref, adj_ref,
              h1_ref, h2_ref, sc_ref, f_scr, h_scr, ps_scr, s_land, sem):
    i = pl.program_id(0)

    @pl.when(i == 0)
    def _():
        wt = wt_ref[...]
        cp1 = pltpu.make_async_copy(s1_ref, s_land, sem)
        cp1.start()
        cp1.wait()
        f_scr[:, :_F] = jnp.dot(s_land[...], wt,
                                preferred_element_type=jnp.float32)
        cp2 = pltpu.make_async_copy(s2_ref, s_land, sem)
        cp2.start()
        cp2.wait()
        f_scr[:, _F:] = jnp.dot(s_land[...], wt,
                                preferred_element_type=jnp.float32)
        ps_scr[...] = jnp.zeros_like(ps_scr)

    acc = jnp.dot(adj_ref[...], f_scr[...],
                  precision=jax.lax.Precision.DEFAULT,
                  preferred_element_type=jnp.float32)
    acc = acc + b_ref[...]
    h = jnp.where(acc > 0, acc, acc * pw_ref[...])
    h1 = h[:, :_F]
    h1_ref[0] = h1
    h2_ref[0] = h[:, _F:]
    h_scr[pl.ds(i * _R, _R), :] = h.astype(jnp.bfloat16)
    ps_scr[...] += jnp.sum(h1, axis=0, keepdims=True)

    @pl.when(i == _NB - 1)
    def _():
        c = jax.nn.sigmoid(ps_scr[...] * (1.0 / _N))
        vcol = jax.lax.dot_general(wb_ref[...], c, (((1,), (1,)), ((), ())),
                                   preferred_element_type=jnp.float32
                                   ).astype(jnp.bfloat16)
        sc_ref[:, 0:1] = jnp.dot(h_scr[:, :_F], vcol,
                                 preferred_element_type=jnp.float32) + bb_ref[...]
        sc_ref[:, 1:2] = jnp.dot(h_scr[:, _F:], vcol,
                                 preferred_element_type=jnp.float32) + bb_ref[...]


def kernel(cc_label, seq1, seq2, adj, sparse, msk, samp_bias1, samp_bias2,
           W_fc, b_gcn, prelu_w, W_bil, b_bil):
    s1 = seq1[0]
    s2 = seq2[0]
    A = adj[0]
    wt = W_fc.T
    b2 = jnp.concatenate([b_gcn, b_gcn])[None, :]
    pw = prelu_w.reshape(1, 1)
    bb = b_bil.reshape(1, 1)

    h1, h2, sc = pl.pallas_call(
        _dgi_body,
        grid=(_NB,),
        in_specs=[
            pl.BlockSpec(memory_space=pl.MemorySpace.ANY),
            pl.BlockSpec(memory_space=pl.MemorySpace.ANY),
            pl.BlockSpec((_F, _F), lambda i: (0, 0)),
            pl.BlockSpec((1, 2 * _F), lambda i: (0, 0)),
            pl.BlockSpec((1, 1), lambda i: (0, 0)),
            pl.BlockSpec((_F, _F), lambda i: (0, 0)),
            pl.BlockSpec((1, 1), lambda i: (0, 0)),
            pl.BlockSpec((_R, _N), lambda i: (i, 0)),
        ],
        out_specs=[
            pl.BlockSpec((1, _R, _F), lambda i: (0, i, 0)),
            pl.BlockSpec((1, _R, _F), lambda i: (0, i, 0)),
            pl.BlockSpec((_N, 2), lambda i: (0, 0)),
        ],
        out_shape=[
            jax.ShapeDtypeStruct((1, _N, _F), jnp.float32),
            jax.ShapeDtypeStruct((1, _N, _F), jnp.float32),
            jax.ShapeDtypeStruct((_N, 2), jnp.float32),
        ],
        scratch_shapes=[
            pltpu.VMEM((_N, 2 * _F), jnp.float32),
            pltpu.VMEM((_N, 2 * _F), jnp.bfloat16),
            pltpu.VMEM((1, _F), jnp.float32),
            pltpu.VMEM((_N, _F), jnp.float32),
            pltpu.SemaphoreType.DMA,
        ],
    )(s1, s2, wt, b2, pw, W_bil[0], bb, A)

    ret = jnp.concatenate([sc[:, 0][None, :] + samp_bias1,
                           sc[:, 1][None, :] + samp_bias2], axis=1)
    return (ret, h1, h2)


# speedup vs baseline: 1.0067x; 1.0060x over previous
import jax
import jax.numpy as jnp
from jax.experimental import pallas as pl
from jax.experimental.pallas import tpu as pltpu

_N = 10000
_F = 128
_R = 200
_NB = _N // _R


def _dgi_body(s1_ref, s2_ref, wt_ref, b_ref, pw_ref, wb_ref, bb_ref, adj_ref,
              h1_ref, h2_ref, sc_ref, f_scr, h_scr, ps_scr, s_land, sem):
    i = pl.program_id(0)

    @pl.when(i == 0)
    def _():
        wt = wt_ref[...]
        cp1 = pltpu.make_async_copy(s1_ref, s_land, sem)
        cp1.start()
        cp1.wait()
        f_scr[:, :_F] = jnp.dot(s_land[...], wt,
                                preferred_element_type=jnp.float32)
        cp2 = pltpu.make_async_copy(s2_ref, s_land, sem)
        cp2.start()
        cp2.wait()
        f_scr[:, _F:] = jnp.dot(s_land[...], wt,
                                preferred_element_type=jnp.float32)
        ps_scr[...] = jnp.zeros_like(ps_scr)

    acc = jnp.dot(adj_ref[...], f_scr[...],
                  precision=jax.lax.Precision.DEFAULT,
                  preferred_element_type=jnp.float32)
    acc = acc + b_ref[...]
    h = jnp.where(acc > 0, acc, acc * pw_ref[...])
    h1 = h[:, :_F]
    h1_ref[0] = h1
    h2_ref[0] = h[:, _F:]
    h_scr[pl.ds(i * _R, _R), :] = h.astype(jnp.bfloat16)
    ps_scr[...] += jnp.sum(h1, axis=0, keepdims=True)

    @pl.when(i == _NB - 1)
    def _():
        c = jax.nn.sigmoid(ps_scr[...] * (1.0 / _N))
        vcol = jax.lax.dot_general(wb_ref[...], c, (((1,), (1,)), ((), ())),
                                   preferred_element_type=jnp.float32
                                   ).astype(jnp.bfloat16)
        sc_ref[:, 0:1] = jnp.dot(h_scr[:, :_F], vcol,
                                 preferred_element_type=jnp.float32) + bb_ref[...]
        sc_ref[:, 1:2] = jnp.dot(h_scr[:, _F:], vcol,
                                 preferred_element_type=jnp.float32) + bb_ref[...]


def kernel(cc_label, seq1, seq2, adj, sparse, msk, samp_bias1, samp_bias2,
           W_fc, b_gcn, prelu_w, W_bil, b_bil):
    s1 = seq1[0]
    s2 = seq2[0]
    A = adj[0]
    wt = W_fc.T
    b2 = jnp.concatenate([b_gcn, b_gcn])[None, :]
    pw = prelu_w.reshape(1, 1)
    bb = b_bil.reshape(1, 1)

    h1, h2, sc = pl.pallas_call(
        _dgi_body,
        grid=(_NB,),
        in_specs=[
            pl.BlockSpec(memory_space=pl.MemorySpace.ANY),
            pl.BlockSpec(memory_space=pl.MemorySpace.ANY),
            pl.BlockSpec((_F, _F), lambda i: (0, 0)),
            pl.BlockSpec((1, 2 * _F), lambda i: (0, 0)),
            pl.BlockSpec((1, 1), lambda i: (0, 0)),
            pl.BlockSpec((_F, _F), lambda i: (0, 0)),
            pl.BlockSpec((1, 1), lambda i: (0, 0)),
            pl.BlockSpec((_R, _N), lambda i: (i, 0)),
        ],
        out_specs=[
            pl.BlockSpec((1, _R, _F), lambda i: (0, i, 0)),
            pl.BlockSpec((1, _R, _F), lambda i: (0, i, 0)),
            pl.BlockSpec((_N, 2), lambda i: (0, 0)),
        ],
        out_shape=[
            jax.ShapeDtypeStruct((1, _N, _F), jnp.float32),
            jax.ShapeDtypeStruct((1, _N, _F), jnp.float32),
            jax.ShapeDtypeStruct((_N, 2), jnp.float32),
        ],
        scratch_shapes=[
            pltpu.VMEM((_N, 2 * _F), jnp.float32),
            pltpu.VMEM((_N, 2 * _F), jnp.bfloat16),
            pltpu.VMEM((1, _F), jnp.float32),
            pltpu.VMEM((_N, _F), jnp.float32),
            pltpu.SemaphoreType.DMA,
        ],
    )(s1, s2, wt, b2, pw, W_bil[0], bb, A)

    ret = jnp.concatenate([sc[:, 0][None, :] + samp_bias1,
                           sc[:, 1][None, :] + samp_bias2], axis=1)
    return (ret, h1, h2)


# speedup vs baseline: 1.0153x; 1.0085x over previous
import jax
import jax.numpy as jnp
from jax.experimental import pallas as pl
from jax.experimental.pallas import tpu as pltpu

_N = 10000
_F = 128
_R = 200
_NB = _N // _R


def _dgi_body(s1_ref, s2_ref, wt_ref, b_ref, pw_ref, wb_ref, bb_ref, adj_ref,
              h1_ref, h2_ref, sc_ref, f_scr, h_scr, ps_scr, s_land, sem):
    i = pl.program_id(0)

    @pl.when(i == 0)
    def _():
        wt = wt_ref[...]
        cp1 = pltpu.make_async_copy(s1_ref, s_land, sem)
        cp1.start()
        cp1.wait()
        f_scr[:, :_F] = jnp.dot(s_land[...], wt,
                                preferred_element_type=jnp.float32)
        cp2 = pltpu.make_async_copy(s2_ref, s_land, sem)
        cp2.start()
        cp2.wait()
        f_scr[:, _F:] = jnp.dot(s_land[...], wt,
                                preferred_element_type=jnp.float32)
        ps_scr[...] = jnp.zeros_like(ps_scr)

    acc = jnp.dot(adj_ref[...], f_scr[...],
                  precision=jax.lax.Precision.DEFAULT,
                  preferred_element_type=jnp.float32)
    acc = acc + b_ref[...]
    h = jnp.where(acc > 0, acc, acc * pw_ref[...])
    h1 = h[:, :_F]
    h1_ref[0] = h1
    h2_ref[0] = h[:, _F:]
    h_scr[pl.ds(i * _R, _R), :] = h
    ps_scr[...] += jnp.sum(h1, axis=0, keepdims=True)

    @pl.when(i == _NB - 1)
    def _():
        c = jax.nn.sigmoid(ps_scr[...] * (1.0 / _N))
        vcol = jax.lax.dot_general(wb_ref[...], c, (((1,), (1,)), ((), ())),
                                   preferred_element_type=jnp.float32)
        sc_ref[:, 0:1] = jnp.dot(h_scr[:, :_F], vcol,
                                 preferred_element_type=jnp.float32) + bb_ref[...]
        sc_ref[:, 1:2] = jnp.dot(h_scr[:, _F:], vcol,
                                 preferred_element_type=jnp.float32) + bb_ref[...]


def kernel(cc_label, seq1, seq2, adj, sparse, msk, samp_bias1, samp_bias2,
           W_fc, b_gcn, prelu_w, W_bil, b_bil):
    s1 = seq1[0]
    s2 = seq2[0]
    A = adj[0]
    wt = W_fc.T
    b2 = jnp.concatenate([b_gcn, b_gcn])[None, :]
    pw = prelu_w.reshape(1, 1)
    bb = b_bil.reshape(1, 1)

    h1, h2, sc = pl.pallas_call(
        _dgi_body,
        grid=(_NB,),
        in_specs=[
            pl.BlockSpec(memory_space=pl.MemorySpace.ANY),
            pl.BlockSpec(memory_space=pl.MemorySpace.ANY),
            pl.BlockSpec((_F, _F), lambda i: (0, 0)),
            pl.BlockSpec((1, 2 * _F), lambda i: (0, 0)),
            pl.BlockSpec((1, 1), lambda i: (0, 0)),
            pl.BlockSpec((_F, _F), lambda i: (0, 0)),
            pl.BlockSpec((1, 1), lambda i: (0, 0)),
            pl.BlockSpec((_R, _N), lambda i: (i, 0)),
        ],
        out_specs=[
            pl.BlockSpec((1, _R, _F), lambda i: (0, i, 0)),
            pl.BlockSpec((1, _R, _F), lambda i: (0, i, 0)),
            pl.BlockSpec((_N, 2), lambda i: (0, 0)),
        ],
        out_shape=[
            jax.ShapeDtypeStruct((1, _N, _F), jnp.float32),
            jax.ShapeDtypeStruct((1, _N, _F), jnp.float32),
            jax.ShapeDtypeStruct((_N, 2), jnp.float32),
        ],
        scratch_shapes=[
            pltpu.VMEM((_N, 2 * _F), jnp.float32),
            pltpu.VMEM((_N, 2 * _F), jnp.float32),
            pltpu.VMEM((1, _F), jnp.float32),
            pltpu.VMEM((_N, _F), jnp.float32),
            pltpu.SemaphoreType.DMA,
        ],
    )(s1, s2, wt, b2, pw, W_bil[0], bb, A)

    ret = jnp.concatenate([sc[:, 0][None, :] + samp_bias1,
                           sc[:, 1][None, :] + samp_bias2], axis=1)
    return (ret, h1, h2)


# speedup vs baseline: 1.0334x; 1.0178x over previous
import jax
import jax.numpy as jnp
from jax.experimental import pallas as pl
from jax.experimental.pallas import tpu as pltpu

_N = 10000
_F = 128
_R = 200
_NB = _N // _R
_B = 3


def _dgi_body(s1_ref, s2_ref, wt_ref, b_ref, pw_ref, wb_ref, bb_ref, adj_ref,
              h1_ref, h2_ref, sc_ref, f_scr, h_scr, ps_scr, s_land, abuf,
              sem, asem):
    i = pl.program_id(0)

    def adj_cp(j, slot):
        return pltpu.make_async_copy(adj_ref.at[pl.ds(j * _R, _R), :],
                                     abuf.at[slot], asem.at[slot])

    @pl.when(i == 0)
    def _():
        adj_cp(0, 0).start()
        adj_cp(1, 1).start()
        wt = wt_ref[...]
        cp1 = pltpu.make_async_copy(s1_ref, s_land, sem)
        cp1.start()
        cp1.wait()
        f_scr[:, :_F] = jnp.dot(s_land[...], wt,
                                preferred_element_type=jnp.float32)
        cp2 = pltpu.make_async_copy(s2_ref, s_land, sem)
        cp2.start()
        cp2.wait()
        f_scr[:, _F:] = jnp.dot(s_land[...], wt,
                                preferred_element_type=jnp.float32)
        ps_scr[...] = jnp.zeros_like(ps_scr)

    @pl.when(i + 2 < _NB)
    def _():
        adj_cp(i + 2, jax.lax.rem(i + 2, _B)).start()

    slot = jax.lax.rem(i, _B)
    adj_cp(i, slot).wait()
    acc = jnp.dot(abuf[slot], f_scr[...],
                  precision=jax.lax.Precision.DEFAULT,
                  preferred_element_type=jnp.float32)
    acc = acc + b_ref[...]
    h = jnp.where(acc > 0, acc, acc * pw_ref[...])
    h1 = h[:, :_F]
    h1_ref[0] = h1
    h2_ref[0] = h[:, _F:]
    h_scr[pl.ds(i * _R, _R), :] = h
    ps_scr[...] += jnp.sum(h1, axis=0, keepdims=True)

    @pl.when(i == _NB - 1)
    def _():
        c = jax.nn.sigmoid(ps_scr[...] * (1.0 / _N))
        vcol = jax.lax.dot_general(wb_ref[...], c, (((1,), (1,)), ((), ())),
                                   preferred_element_type=jnp.float32)
        sc_ref[:, 0:1] = jnp.dot(h_scr[:, :_F], vcol,
                                 preferred_element_type=jnp.float32) + bb_ref[...]
        sc_ref[:, 1:2] = jnp.dot(h_scr[:, _F:], vcol,
                                 preferred_element_type=jnp.float32) + bb_ref[...]


def kernel(cc_label, seq1, seq2, adj, sparse, msk, samp_bias1, samp_bias2,
           W_fc, b_gcn, prelu_w, W_bil, b_bil):
    s1 = seq1[0]
    s2 = seq2[0]
    A = adj[0]
    wt = W_fc.T
    b2 = jnp.concatenate([b_gcn, b_gcn])[None, :]
    pw = prelu_w.reshape(1, 1)
    bb = b_bil.reshape(1, 1)

    h1, h2, sc = pl.pallas_call(
        _dgi_body,
        grid=(_NB,),
        in_specs=[
            pl.BlockSpec(memory_space=pl.MemorySpace.ANY),
            pl.BlockSpec(memory_space=pl.MemorySpace.ANY),
            pl.BlockSpec((_F, _F), lambda i: (0, 0)),
            pl.BlockSpec((1, 2 * _F), lambda i: (0, 0)),
            pl.BlockSpec((1, 1), lambda i: (0, 0)),
            pl.BlockSpec((_F, _F), lambda i: (0, 0)),
            pl.BlockSpec((1, 1), lambda i: (0, 0)),
            pl.BlockSpec(memory_space=pl.MemorySpace.ANY),
        ],
        out_specs=[
            pl.BlockSpec((1, _R, _F), lambda i: (0, i, 0)),
            pl.BlockSpec((1, _R, _F), lambda i: (0, i, 0)),
            pl.BlockSpec((_N, 2), lambda i: (0, 0)),
        ],
        out_shape=[
            jax.ShapeDtypeStruct((1, _N, _F), jnp.float32),
            jax.ShapeDtypeStruct((1, _N, _F), jnp.float32),
            jax.ShapeDtypeStruct((_N, 2), jnp.float32),
        ],
        scratch_shapes=[
            pltpu.VMEM((_N, 2 * _F), jnp.float32),
            pltpu.VMEM((_N, 2 * _F), jnp.float32),
            pltpu.VMEM((1, _F), jnp.float32),
            pltpu.VMEM((_N, _F), jnp.float32),
            pltpu.VMEM((_B, _R, _N), jnp.float32),
            pltpu.SemaphoreType.DMA,
            pltpu.SemaphoreType.DMA((_B,)),
        ],
    )(s1, s2, wt, b2, pw, W_bil[0], bb, A)

    ret = jnp.concatenate([sc[:, 0][None, :] + samp_bias1,
                           sc[:, 1][None, :] + samp_bias2], axis=1)
    return (ret, h1, h2)
